# baseline (speedup 1.0000x reference)
import jax
import jax.numpy as jnp
from jax import lax
from jax.experimental import pallas as pl
from jax.experimental.pallas import tpu as pltpu
from jax.experimental.pallas import tpu_sc as plsc

B = 16
N = 4096
G = 256
K = 32
NW = 32
NSPLIT = 2
HB = B // NSPLIT
RPW = HB * G // NW


def _fps_kernel(x_ref, y_ref, z_ref, idx_ref):
    nb = x_ref.shape[0]
    X = x_ref[...]
    Y = y_ref[...]
    Z = z_ref[...]
    iota = lax.broadcasted_iota(jnp.int32, (nb, N), 1)
    giota = lax.broadcasted_iota(jnp.int32, (nb, G), 1)

    def body(t, carry):
        sel, cx, cy, cz, dists = carry
        idx_ref[...] = jnp.where(giota == t, sel, idx_ref[...])
        d = (X - cx) ** 2 + (Y - cy) ** 2 + (Z - cz) ** 2
        dmin = jnp.minimum(dists, d)
        nsel = jnp.argmax(dmin, axis=1, keepdims=True).astype(jnp.int32)
        mask = iota == nsel
        ncx = jnp.sum(jnp.where(mask, X, 0.0), axis=1, keepdims=True)
        ncy = jnp.sum(jnp.where(mask, Y, 0.0), axis=1, keepdims=True)
        ncz = jnp.sum(jnp.where(mask, Z, 0.0), axis=1, keepdims=True)
        return (nsel, ncx, ncy, ncz, dmin)

    init = (jnp.zeros((nb, 1), jnp.int32), X[:, 0:1], Y[:, 0:1], Z[:, 0:1],
            jnp.full((nb, N), 1e10, dtype=jnp.float32))
    lax.fori_loop(0, G, body, init)


def _sc_knn(pxh, pyh, pzh, idxh,
            cxo, cyo, czo, no,
            pxv, pyv, pzv, idxv, cxa, cya, cza,
            d2v, hist, sums, bufb, bufi, bufp, nbv):
    i32 = jnp.int32
    wid = lax.axis_index("s") * 2 + lax.axis_index("c")
    bidx = (wid * RPW) // G
    iota16 = lax.iota(i32, 16)
    ones16 = jnp.ones((16,), i32)
    zeros16 = jnp.zeros((16,), i32)

    pltpu.sync_copy(pxh.at[pl.ds(bidx * N, N)], pxv)
    pltpu.sync_copy(pyh.at[pl.ds(bidx * N, N)], pyv)
    pltpu.sync_copy(pzh.at[pl.ds(bidx * N, N)], pzv)
    pltpu.sync_copy(idxh.at[pl.ds(wid * RPW, RPW)], idxv)

    for q in range(RPW // 16):
        iq = idxv[pl.ds(q * 16, 16)]
        cxa[pl.ds(q * 16, 16)] = plsc.load_gather(pxv, [iq])
        cya[pl.ds(q * 16, 16)] = plsc.load_gather(pyv, [iq])
        cza[pl.ds(q * 16, 16)] = plsc.load_gather(pzv, [iq])
    pltpu.sync_copy(cxa, cxo.at[pl.ds(wid * RPW, RPW)])
    pltpu.sync_copy(cya, cyo.at[pl.ds(wid * RPW, RPW)])
    pltpu.sync_copy(cza, czo.at[pl.ds(wid * RPW, RPW)])

    def _splat(x):
        return x if getattr(x, "shape", ()) == (16,) else jnp.broadcast_to(x, (16,))

    def _pick(vec, i16):
        sums[...] = vec
        return plsc.load_gather(sums, [i16])

    def _hist_scan(target16):
        acc = zeros16
        for i in range(16):
            acc = acc + plsc.load_gather(hist, [iota16 * 16 + i])
        cs = plsc.cumsum(acc)
        tch = _splat(plsc.all_reduce_ffs(cs >= target16))
        cb = _pick(cs - acc, tch)
        v = plsc.load_gather(hist, [tch * 16 + iota16])
        cs2 = plsc.cumsum(v)
        lane = _splat(plsc.all_reduce_ffs((cb + cs2) >= target16))
        bucket = tch * 16 + lane
        cum_below = cb + _pick(cs2, lane) - _pick(v, lane)
        return bucket, cum_below

    def row(j, _):
        j16 = jnp.broadcast_to(j, (16,))
        cxb = plsc.load_gather(cxa, [j16])
        cyb = plsc.load_gather(cya, [j16])
        czb = plsc.load_gather(cza, [j16])

        for t in range(16):
            hist[pl.ds(t * 16, 16)] = zeros16

        @plsc.parallel_loop(0, N // 16, unroll=16)
        def c1(c):
            o = c * 16
            dx = cxb - pxv[pl.ds(o, 16)]
            dy = cyb - pyv[pl.ds(o, 16)]
            dz = czb - pzv[pl.ds(o, 16)]
            d = (dx * dx + dy * dy) + dz * dz
            d2v[pl.ds(o, 16)] = d
            bb = jnp.right_shift(plsc.bitcast(d, i32), 23)
            plsc.addupdate_scatter(hist, [bb], ones16)

        b1, c_below1 = _hist_scan(jnp.full((16,), K, i32))

        for t in range(16):
            hist[pl.ds(t * 16, 16)] = zeros16

        @plsc.parallel_loop(0, N // 16, unroll=8, carry=zeros16)
        def c2(c, w2):
            o = c * 16
            bits = plsc.bitcast(d2v[pl.ds(o, 16)], i32)
            e = jnp.right_shift(bits, 23)
            pm = e == b1
            bb = jnp.bitwise_and(jnp.right_shift(bits, 15), 0xFF)
            plsc.addupdate_scatter(hist, [bb], ones16, mask=pm)
            ple = e <= b1
            cl = plsc.cumsum(ple.astype(i32))
            pos = (w2 + cl) - 1
            plsc.store_scatter(bufp, [pos], o + iota16, mask=ple)
            return w2 + plsc.all_reduce_population_count(ple)
        w2 = jnp.max(c2)

        b2, _cb2 = _hist_scan(jnp.full((16,), K, i32) - c_below1)
        prefix = b1 * 256 + b2

        @plsc.parallel_loop(0, jnp.right_shift(w2 + 15, 4), unroll=4,
                            carry=zeros16)
        def c3(c, w):
            o = c * 16
            mvalid = (o + iota16) < w2
            pos = bufp[pl.ds(o, 16)]
            dv = plsc.load_gather(d2v, [pos], mask=mvalid)
            bits = plsc.bitcast(dv, i32)
            msel = mvalid & (jnp.right_shift(bits, 15) <= prefix)
            cl = plsc.cumsum(msel.astype(i32))
            p2 = (w + cl) - 1
            plsc.store_scatter(bufb, [p2], bits, mask=msel)
            plsc.store_scatter(bufi, [p2], pos, mask=msel)
            return w + plsc.all_reduce_population_count(msel)
        w = jnp.max(c3)

        def rankchunk(ci, carry):
            o = ci * 16
            bv = bufb[pl.ds(o, 16)]
            iv = bufi[pl.ds(o, 16)]
            mvalid = (o + iota16) < w

            @plsc.parallel_loop(0, w, unroll=4, carry=zeros16)
            def inner(e, r):
                e16 = jnp.broadcast_to(e, (16,))
                be = plsc.load_gather(bufb, [e16])
                ie = plsc.load_gather(bufi, [e16])
                less = (be < bv) | ((be == bv) & (ie < iv))
                return r + less.astype(i32)
            rank = inner
            msel = mvalid & (rank < K)
            xg = plsc.load_gather(pxv, [iv], mask=msel)
            yg = plsc.load_gather(pyv, [iv], mask=msel)
            zg = plsc.load_gather(pzv, [iv], mask=msel)
            rank3 = rank + rank + rank
            plsc.store_scatter(nbv, [j16, rank3], xg - cxb, mask=msel)
            plsc.store_scatter(nbv, [j16, rank3 + 1], yg - cyb, mask=msel)
            plsc.store_scatter(nbv, [j16, rank3 + 2], zg - czb, mask=msel)
            return carry
        lax.fori_loop(0, (w + 15) // 16, rankchunk, 0)
        return _

    lax.fori_loop(0, RPW, row, 0)

    pltpu.sync_copy(nbv, no.at[pl.ds(wid * RPW, RPW)])


@jax.jit
def kernel(xyz, data_3d):
    x = xyz[:, :, 0]
    y = xyz[:, :, 1]
    z = xyz[:, :, 2]
    px = data_3d[:, :, 0]
    py = data_3d[:, :, 1]
    pz = data_3d[:, :, 2]

    mesh = plsc.VectorSubcoreMesh(core_axis_name="c", subcore_axis_name="s")
    knn = pl.kernel(
        _sc_knn,
        out_type=[
            jax.ShapeDtypeStruct((HB * G,), jnp.float32),
            jax.ShapeDtypeStruct((HB * G,), jnp.float32),
            jax.ShapeDtypeStruct((HB * G,), jnp.float32),
            jax.ShapeDtypeStruct((HB * G, 3 * K), jnp.float32),
        ],
        mesh=mesh,
        compiler_params=pltpu.CompilerParams(needs_layout_passes=False),
        scratch_types=[
            pltpu.VMEM((N,), jnp.float32),
            pltpu.VMEM((N,), jnp.float32),
            pltpu.VMEM((N,), jnp.float32),
            pltpu.VMEM((RPW,), jnp.int32),
            pltpu.VMEM((RPW,), jnp.float32),
            pltpu.VMEM((RPW,), jnp.float32),
            pltpu.VMEM((RPW,), jnp.float32),
            pltpu.VMEM((N,), jnp.float32),
            pltpu.VMEM((256,), jnp.int32),
            pltpu.VMEM((16,), jnp.int32),
            pltpu.VMEM((N,), jnp.int32),
            pltpu.VMEM((N,), jnp.int32),
            pltpu.VMEM((N,), jnp.int32),
            pltpu.VMEM((RPW, 3 * K), jnp.float32),
        ],
    )

    parts = []
    for h in range(NSPLIT):
        s = slice(h * HB, (h + 1) * HB)
        (selidx,) = pl.pallas_call(
            _fps_kernel,
            out_shape=[jax.ShapeDtypeStruct((HB, G), jnp.int32)],
        )(x[s], y[s], z[s])
        parts.append(knn(px[s].reshape(-1), py[s].reshape(-1),
                         pz[s].reshape(-1), selidx.reshape(-1)))

    cxo, cyo, czo = (jnp.concatenate([p[i] for p in parts], axis=0)
                     for i in range(3))
    center = jnp.stack([cxo.reshape(B, G), cyo.reshape(B, G),
                        czo.reshape(B, G)], axis=-1)
    neighborhood = jnp.concatenate(
        [p[3].reshape(HB, G, K, 3) for p in parts], axis=0)
    return neighborhood, center


# speedup vs baseline: 1.0108x; 1.0108x over previous
import jax
import jax.numpy as jnp
from jax import lax
from jax.experimental import pallas as pl
from jax.experimental.pallas import tpu as pltpu
from jax.experimental.pallas import tpu_sc as plsc

B = 16
N = 4096
G = 256
K = 32
NW = 32
NSPLIT = 2
HB = B // NSPLIT
RPW = HB * G // NW


def _fps_kernel(x_ref, y_ref, z_ref, idx_ref):
    nb = x_ref.shape[0]
    X = x_ref[...]
    Y = y_ref[...]
    Z = z_ref[...]
    iota = lax.broadcasted_iota(jnp.int32, (nb, N), 1)
    giota = lax.broadcasted_iota(jnp.int32, (nb, G), 1)

    def body(t, carry):
        sel, cx, cy, cz, dists = carry
        idx_ref[...] = jnp.where(giota == t, sel, idx_ref[...])
        d = (X - cx) ** 2 + (Y - cy) ** 2 + (Z - cz) ** 2
        dmin = jnp.minimum(dists, d)
        nsel = jnp.argmax(dmin, axis=1, keepdims=True).astype(jnp.int32)
        mask = iota == nsel
        ncx = jnp.sum(jnp.where(mask, X, 0.0), axis=1, keepdims=True)
        ncy = jnp.sum(jnp.where(mask, Y, 0.0), axis=1, keepdims=True)
        ncz = jnp.sum(jnp.where(mask, Z, 0.0), axis=1, keepdims=True)
        return (nsel, ncx, ncy, ncz, dmin)

    init = (jnp.zeros((nb, 1), jnp.int32), X[:, 0:1], Y[:, 0:1], Z[:, 0:1],
            jnp.full((nb, N), 1e10, dtype=jnp.float32))
    lax.fori_loop(0, G, body, init)


def _sc_knn(pxh, pyh, pzh, idxh,
            cxo, cyo, czo, no,
            pxv, pyv, pzv, idxv, cxa, cya, cza,
            d2v, hist, sums, bufb, bufi, bufp, nbv):
    i32 = jnp.int32
    wid = lax.axis_index("s") * 2 + lax.axis_index("c")
    bidx = (wid * RPW) // G
    iota16 = lax.iota(i32, 16)
    ones16 = jnp.ones((16,), i32)
    zeros16 = jnp.zeros((16,), i32)

    pltpu.sync_copy(pxh.at[pl.ds(bidx * N, N)], pxv)
    pltpu.sync_copy(pyh.at[pl.ds(bidx * N, N)], pyv)
    pltpu.sync_copy(pzh.at[pl.ds(bidx * N, N)], pzv)
    pltpu.sync_copy(idxh.at[pl.ds(wid * RPW, RPW)], idxv)

    for q in range(RPW // 16):
        iq = idxv[pl.ds(q * 16, 16)]
        cxa[pl.ds(q * 16, 16)] = plsc.load_gather(pxv, [iq])
        cya[pl.ds(q * 16, 16)] = plsc.load_gather(pyv, [iq])
        cza[pl.ds(q * 16, 16)] = plsc.load_gather(pzv, [iq])
    pltpu.sync_copy(cxa, cxo.at[pl.ds(wid * RPW, RPW)])
    pltpu.sync_copy(cya, cyo.at[pl.ds(wid * RPW, RPW)])
    pltpu.sync_copy(cza, czo.at[pl.ds(wid * RPW, RPW)])

    def _splat(x):
        return x if getattr(x, "shape", ()) == (16,) else jnp.broadcast_to(x, (16,))

    def _pick(vec, i16):
        sums[...] = vec
        return plsc.load_gather(sums, [i16])

    def _hist_scan(target16):
        acc = zeros16
        for i in range(16):
            acc = acc + plsc.load_gather(hist, [iota16 * 16 + i])
        cs = plsc.cumsum(acc)
        tch = _splat(plsc.all_reduce_ffs(cs >= target16))
        cb = _pick(cs - acc, tch)
        v = plsc.load_gather(hist, [tch * 16 + iota16])
        cs2 = plsc.cumsum(v)
        lane = _splat(plsc.all_reduce_ffs((cb + cs2) >= target16))
        bucket = tch * 16 + lane
        cum_below = cb + _pick(cs2, lane) - _pick(v, lane)
        return bucket, cum_below

    def row(j, _):
        j16 = jnp.broadcast_to(j, (16,))
        cxb = plsc.load_gather(cxa, [j16])
        cyb = plsc.load_gather(cya, [j16])
        czb = plsc.load_gather(cza, [j16])

        for t in range(16):
            hist[pl.ds(t * 16, 16)] = zeros16

        @plsc.parallel_loop(0, N // 16, unroll=8)
        def c1(c):
            o = c * 16
            dx = cxb - pxv[pl.ds(o, 16)]
            dy = cyb - pyv[pl.ds(o, 16)]
            dz = czb - pzv[pl.ds(o, 16)]
            d = (dx * dx + dy * dy) + dz * dz
            d2v[pl.ds(o, 16)] = d
            bb = jnp.right_shift(plsc.bitcast(d, i32), 23)
            plsc.addupdate_scatter(hist, [bb], ones16)

        b1, c_below1 = _hist_scan(jnp.full((16,), K, i32))

        for t in range(16):
            hist[pl.ds(t * 16, 16)] = zeros16

        @plsc.parallel_loop(0, N // 16, unroll=4, carry=zeros16)
        def c2(c, w2):
            o = c * 16
            bits = plsc.bitcast(d2v[pl.ds(o, 16)], i32)
            e = jnp.right_shift(bits, 23)
            pm = e == b1
            bb = jnp.bitwise_and(jnp.right_shift(bits, 15), 0xFF)
            plsc.addupdate_scatter(hist, [bb], ones16, mask=pm)
            ple = e <= b1
            cl = plsc.cumsum(ple.astype(i32))
            pos = (w2 + cl) - 1
            plsc.store_scatter(bufp, [pos], o + iota16, mask=ple)
            return w2 + plsc.all_reduce_population_count(ple)
        w2 = jnp.max(c2)

        b2, _cb2 = _hist_scan(jnp.full((16,), K, i32) - c_below1)
        prefix = b1 * 256 + b2

        @plsc.parallel_loop(0, jnp.right_shift(w2 + 15, 4), unroll=4,
                            carry=zeros16)
        def c3(c, w):
            o = c * 16
            mvalid = (o + iota16) < w2
            pos = bufp[pl.ds(o, 16)]
            dv = plsc.load_gather(d2v, [pos], mask=mvalid)
            bits = plsc.bitcast(dv, i32)
            msel = mvalid & (jnp.right_shift(bits, 15) <= prefix)
            cl = plsc.cumsum(msel.astype(i32))
            p2 = (w + cl) - 1
            plsc.store_scatter(bufb, [p2], bits, mask=msel)
            plsc.store_scatter(bufi, [p2], pos, mask=msel)
            return w + plsc.all_reduce_population_count(msel)
        w = jnp.max(c3)

        def rankchunk(ci, carry):
            o = ci * 16
            bv = bufb[pl.ds(o, 16)]
            iv = bufi[pl.ds(o, 16)]
            mvalid = (o + iota16) < w

            @plsc.parallel_loop(0, w, unroll=4, carry=zeros16)
            def inner(e, r):
                e16 = jnp.broadcast_to(e, (16,))
                be = plsc.load_gather(bufb, [e16])
                ie = plsc.load_gather(bufi, [e16])
                less = (be < bv) | ((be == bv) & (ie < iv))
                return r + less.astype(i32)
            rank = inner
            msel = mvalid & (rank < K)
            xg = plsc.load_gather(pxv, [iv], mask=msel)
            yg = plsc.load_gather(pyv, [iv], mask=msel)
            zg = plsc.load_gather(pzv, [iv], mask=msel)
            rank3 = rank + rank + rank
            plsc.store_scatter(nbv, [j16, rank3], xg - cxb, mask=msel)
            plsc.store_scatter(nbv, [j16, rank3 + 1], yg - cyb, mask=msel)
            plsc.store_scatter(nbv, [j16, rank3 + 2], zg - czb, mask=msel)
            return carry
        lax.fori_loop(0, (w + 15) // 16, rankchunk, 0)
        return _

    lax.fori_loop(0, RPW, row, 0)

    pltpu.sync_copy(nbv, no.at[pl.ds(wid * RPW, RPW)])


@jax.jit
def kernel(xyz, data_3d):
    x = xyz[:, :, 0]
    y = xyz[:, :, 1]
    z = xyz[:, :, 2]
    px = data_3d[:, :, 0]
    py = data_3d[:, :, 1]
    pz = data_3d[:, :, 2]

    mesh = plsc.VectorSubcoreMesh(core_axis_name="c", subcore_axis_name="s")
    knn = pl.kernel(
        _sc_knn,
        out_type=[
            jax.ShapeDtypeStruct((HB * G,), jnp.float32),
            jax.ShapeDtypeStruct((HB * G,), jnp.float32),
            jax.ShapeDtypeStruct((HB * G,), jnp.float32),
            jax.ShapeDtypeStruct((HB * G, 3 * K), jnp.float32),
        ],
        mesh=mesh,
        compiler_params=pltpu.CompilerParams(needs_layout_passes=False),
        scratch_types=[
            pltpu.VMEM((N,), jnp.float32),
            pltpu.VMEM((N,), jnp.float32),
            pltpu.VMEM((N,), jnp.float32),
            pltpu.VMEM((RPW,), jnp.int32),
            pltpu.VMEM((RPW,), jnp.float32),
            pltpu.VMEM((RPW,), jnp.float32),
            pltpu.VMEM((RPW,), jnp.float32),
            pltpu.VMEM((N,), jnp.float32),
            pltpu.VMEM((256,), jnp.int32),
            pltpu.VMEM((16,), jnp.int32),
            pltpu.VMEM((N,), jnp.int32),
            pltpu.VMEM((N,), jnp.int32),
            pltpu.VMEM((N,), jnp.int32),
            pltpu.VMEM((RPW, 3 * K), jnp.float32),
        ],
    )

    parts = []
    for h in range(NSPLIT):
        s = slice(h * HB, (h + 1) * HB)
        (selidx,) = pl.pallas_call(
            _fps_kernel,
            out_shape=[jax.ShapeDtypeStruct((HB, G), jnp.int32)],
        )(x[s], y[s], z[s])
        parts.append(knn(px[s].reshape(-1), py[s].reshape(-1),
                         pz[s].reshape(-1), selidx.reshape(-1)))

    cxo, cyo, czo = (jnp.concatenate([p[i] for p in parts], axis=0)
                     for i in range(3))
    center = jnp.stack([cxo.reshape(B, G), cyo.reshape(B, G),
                        czo.reshape(B, G)], axis=-1)
    neighborhood = jnp.concatenate(
        [p[3].reshape(HB, G, K, 3) for p in parts], axis=0)
    return neighborhood, center
